# baseline (speedup 1.0000x reference)
import jax
import jax.numpy as jnp
from jax.experimental import pallas as pl


_BLOCK = 10000


def _body(x_ref, o_ref):
    o_ref[...] = x_ref[...]

    @pl.when(pl.program_id(0) == 0)
    def _():
        o_ref[0:2, :] = jnp.ones((2, o_ref.shape[1]), o_ref.dtype)


def kernel(x):
    n, d = x.shape
    return pl.pallas_call(
        _body,
        grid=(n // _BLOCK,),
        in_specs=[pl.BlockSpec((_BLOCK, d), lambda i: (i, 0))],
        out_specs=pl.BlockSpec((_BLOCK, d), lambda i: (i, 0)),
        out_shape=jax.ShapeDtypeStruct((n, d), x.dtype),
    )(x)
